# baseline (speedup 1.0000x reference)
import functools

import jax
import jax.numpy as jnp
import numpy as np
from jax.experimental import pallas as pl
from jax.experimental.pallas import tpu as pltpu
from jax.experimental.shard_map import shard_map
from jax.sharding import Mesh, PartitionSpec as P

B, N, F = 4, 2048, 512
H1, H2, L = 512, 512, 128
_EPS = 1e-5
_CHUNK = 512
_NCHUNKS = N // _CHUNK


def _norm_bf16(x, bias_free=True):
    mean = jnp.mean(x, axis=-1, keepdims=True)
    xc = x - mean
    var = jnp.mean(xc * xc, axis=-1, keepdims=True)
    return (xc * jax.lax.rsqrt(var + _EPS)).astype(jnp.bfloat16)


def _bf16_dot(a_bf, b_bf):
    return jax.lax.dot_general(
        a_bf, b_bf, (((1,), (0,)), ((), ())),
        preferred_element_type=jnp.float32)


def _gcn_body(v_ref, adj_ref, w1g_ref, bw1_ref, w2g_ref, bw2_ref,
              wout_ref, bout_ref, out_ref, s2b_ref):
    x = v_ref[0]

    xn = _norm_bf16(x)
    s1b = (_bf16_dot(xn, w1g_ref[...]) + bw1_ref[...]).astype(jnp.bfloat16)

    for i in range(_NCHUNKS):
        rows = pl.ds(i * _CHUNK, _CHUNK)
        adjc = adj_ref[0, rows, :].astype(jnp.bfloat16)
        h1c = jnp.maximum(_bf16_dot(adjc, s1b), 0.0)
        x2c = _norm_bf16(h1c)
        s2b_ref[rows, :] = (
            _bf16_dot(x2c, w2g_ref[...]) + bw2_ref[...]).astype(jnp.bfloat16)

    acc = jnp.zeros((1, L), jnp.float32)
    for i in range(_NCHUNKS):
        rows = pl.ds(i * _CHUNK, _CHUNK)
        adjc = adj_ref[0, rows, :].astype(jnp.bfloat16)
        h2c = jnp.maximum(_bf16_dot(adjc, s2b_ref[...]), 0.0)
        src = jnp.sum(h2c, axis=-1)[None, :]
        acc = acc + jax.lax.dot_general(
            src, wout_ref[rows, :], (((1,), (0,)), ((), ())),
            preferred_element_type=jnp.float32,
            precision=jax.lax.Precision.HIGHEST)
    out_ref[0] = acc + bout_ref[...]


def _gcn_local(v, adj, w1g, bw1, w2g, bw2, wout, bo):
    b_local = v.shape[0]
    batch_spec = lambda shape: pl.BlockSpec(shape, lambda b: (b,) + (0,) * (len(shape) - 1))
    fixed_spec = lambda shape: pl.BlockSpec(shape, lambda b: (0,) * len(shape))

    out = pl.pallas_call(
        _gcn_body,
        grid=(b_local,),
        in_specs=[
            batch_spec((1, N, F)),
            batch_spec((1, N, N)),
            fixed_spec((F, H1)),
            fixed_spec((1, H1)),
            fixed_spec((H1, H2)),
            fixed_spec((1, H2)),
            fixed_spec((N, L)),
            fixed_spec((1, L)),
        ],
        out_specs=pl.BlockSpec((1, 1, L), lambda b: (b, 0, 0)),
        out_shape=jax.ShapeDtypeStruct((b_local, 1, L), jnp.float32),
        scratch_shapes=[pltpu.VMEM((N, H2), jnp.bfloat16)],
        compiler_params=pltpu.CompilerParams(
            dimension_semantics=("arbitrary",),
        ),
    )(v, adj, w1g, bw1, w2g, bw2, wout, bo)
    return out


@functools.partial(jax.jit, static_argnames=())
def kernel(v, adj, gamma1, beta1, W1, gamma2, beta2, W2, W_out, b_out):
    w1g = (gamma1[:, None] * W1).astype(jnp.bfloat16)
    w2g = (gamma2[:, None] * W2).astype(jnp.bfloat16)
    bw1 = (beta1 @ W1).reshape(1, H1)
    bw2 = (beta2 @ W2).reshape(1, H2)
    bo = b_out.reshape(1, L)

    devs = jax.devices()
    nshard = max(d for d in (4, 2, 1) if B % d == 0 and d <= len(devs))
    if nshard > 1:
        mesh = Mesh(np.array(devs[:nshard]), ("b",))
        run = shard_map(
            _gcn_local, mesh=mesh,
            in_specs=(P("b"), P("b"), P(), P(), P(), P(), P(), P()),
            out_specs=P("b"), check_rep=False)
    else:
        run = _gcn_local
    out = run(v, adj, w1g, bw1, w2g, bw2, W_out, bo)
    return out.reshape(B, L)


# speedup vs baseline: 6.5979x; 6.5979x over previous
import functools

import jax
import jax.numpy as jnp
from jax.experimental import pallas as pl
from jax.experimental.pallas import tpu as pltpu

B, N, F = 4, 2048, 512
H1, H2, L = 512, 512, 128
_EPS = 1e-5
_CHUNK = 512
_NCHUNKS = N // _CHUNK


def _norm_bf16(x, bias_free=True):
    mean = jnp.mean(x, axis=-1, keepdims=True)
    xc = x - mean
    var = jnp.mean(xc * xc, axis=-1, keepdims=True)
    return (xc * jax.lax.rsqrt(var + _EPS)).astype(jnp.bfloat16)


def _bf16_dot(a_bf, b_bf):
    return jax.lax.dot_general(
        a_bf, b_bf, (((1,), (0,)), ((), ())),
        preferred_element_type=jnp.float32)


def _gcn_body(v_ref, adj_ref, w1g_ref, bw1_ref, w2g_ref, bw2_ref,
              wout_ref, bout_ref, out_ref, s2b_ref, adjb_ref):
    x = v_ref[0]

    for i in range(_NCHUNKS):
        rows = pl.ds(i * _CHUNK, _CHUNK)
        adjb_ref[rows, :] = adj_ref[0, rows, :].astype(jnp.bfloat16)

    xn = _norm_bf16(x)
    s1b = (_bf16_dot(xn, w1g_ref[...]) + bw1_ref[...]).astype(jnp.bfloat16)

    h1 = jnp.maximum(_bf16_dot(adjb_ref[...], s1b), 0.0)
    x2 = _norm_bf16(h1)
    s2b_ref[...] = (
        _bf16_dot(x2, w2g_ref[...]) + bw2_ref[...]).astype(jnp.bfloat16)

    acc = jnp.zeros((1, L), jnp.float32)
    for i in range(_NCHUNKS):
        rows = pl.ds(i * _CHUNK, _CHUNK)
        h2c = jnp.maximum(_bf16_dot(adjb_ref[rows, :], s2b_ref[...]), 0.0)
        src = jnp.sum(h2c, axis=-1)[None, :]
        acc = acc + jax.lax.dot_general(
            src, wout_ref[rows, :], (((1,), (0,)), ((), ())),
            preferred_element_type=jnp.float32,
            precision=jax.lax.Precision.HIGHEST)
    out_ref[0] = acc + bout_ref[...]


@functools.partial(jax.jit, static_argnames=())
def kernel(v, adj, gamma1, beta1, W1, gamma2, beta2, W2, W_out, b_out):
    w1g = (gamma1[:, None] * W1).astype(jnp.bfloat16)
    w2g = (gamma2[:, None] * W2).astype(jnp.bfloat16)
    bw1 = (beta1 @ W1).reshape(1, H1)
    bw2 = (beta2 @ W2).reshape(1, H2)
    bo = b_out.reshape(1, L)

    grid = (B,)
    batch_spec = lambda shape: pl.BlockSpec(shape, lambda b: (b,) + (0,) * (len(shape) - 1))
    fixed_spec = lambda shape: pl.BlockSpec(shape, lambda b: (0,) * len(shape))

    out = pl.pallas_call(
        _gcn_body,
        grid=grid,
        in_specs=[
            batch_spec((1, N, F)),
            batch_spec((1, N, N)),
            fixed_spec((F, H1)),
            fixed_spec((1, H1)),
            fixed_spec((H1, H2)),
            fixed_spec((1, H2)),
            fixed_spec((N, L)),
            fixed_spec((1, L)),
        ],
        out_specs=pl.BlockSpec((1, 1, L), lambda b: (b, 0, 0)),
        out_shape=jax.ShapeDtypeStruct((B, 1, L), jnp.float32),
        scratch_shapes=[pltpu.VMEM((N, H2), jnp.bfloat16),
                        pltpu.VMEM((N, N), jnp.bfloat16)],
        compiler_params=pltpu.CompilerParams(
            dimension_semantics=("arbitrary",),
        ),
    )(v, adj, w1g, bw1, w2g, bw2, W_out, bo)
    return out.reshape(B, L)
